# baseline (speedup 1.0000x reference)
import functools

import jax
import jax.numpy as jnp
from jax import lax
from jax.experimental import pallas as pl
from jax.experimental.pallas import tpu as pltpu

GRID = 8
GV = GRID ** 3
EPS = 1e-5
B = 32
N = 4096
F = 128


def _mlp_body(pts_ref, w1_ref, b1_ref, w2_ref, b2_ref, feat_ref):
    p = pts_ref[...]
    x = jnp.dot(p, w1_ref[...], preferred_element_type=jnp.float32)
    x = jnp.maximum(x + b1_ref[...], 0.0)
    y = jnp.dot(x, w2_ref[...], preferred_element_type=jnp.float32)
    feat_ref[...] = jnp.maximum(y + b2_ref[...], 0.0)


def _point_mlp(points2d, w1f, b1f, w2f, b2f):
    return pl.pallas_call(
        _mlp_body,
        grid=(B,),
        in_specs=[
            pl.BlockSpec((N, 3), lambda b: (b, 0)),
            pl.BlockSpec((3, 64), lambda b: (0, 0)),
            pl.BlockSpec((1, 64), lambda b: (0, 0)),
            pl.BlockSpec((64, 128), lambda b: (0, 0)),
            pl.BlockSpec((1, 128), lambda b: (0, 0)),
        ],
        out_specs=pl.BlockSpec((N, F), lambda b: (b, 0)),
        out_shape=jax.ShapeDtypeStruct((B * N, F), jnp.float32),
    )(points2d, w1f, b1f, w2f, b2f)


KBLK = 4096
NKB = GV * F // KBLK


def _head_body(gf_ref, wf1_ref, s3_ref, bf1f_ref, wf2s_ref, bf2f_ref,
               wf3_ref, bf3_ref, out_ref, acc_ref):
    k = pl.program_id(0)

    @pl.when(k == 0)
    def _():
        acc_ref[...] = jnp.zeros_like(acc_ref)

    acc_ref[...] += jnp.dot(gf_ref[...], wf1_ref[...],
                            preferred_element_type=jnp.float32)

    @pl.when(k == NKB - 1)
    def _():
        h = jnp.maximum(acc_ref[...] * s3_ref[...] + bf1f_ref[...], 0.0)
        h2 = jnp.dot(h, wf2s_ref[...], preferred_element_type=jnp.float32)
        h2 = jnp.maximum(h2 + bf2f_ref[...], 0.0)
        out_ref[...] = jnp.dot(h2, wf3_ref[...],
                               preferred_element_type=jnp.float32) + bf3_ref[...]


def _head(gf, wf1, s3, bf1f, wf2s, bf2f, wf3, bf3):
    return pl.pallas_call(
        _head_body,
        grid=(NKB,),
        in_specs=[
            pl.BlockSpec((B, KBLK), lambda k: (0, k)),
            pl.BlockSpec((KBLK, 512), lambda k: (k, 0)),
            pl.BlockSpec((1, 512), lambda k: (0, 0)),
            pl.BlockSpec((1, 512), lambda k: (0, 0)),
            pl.BlockSpec((512, 256), lambda k: (0, 0)),
            pl.BlockSpec((1, 256), lambda k: (0, 0)),
            pl.BlockSpec((256, 40), lambda k: (0, 0)),
            pl.BlockSpec((1, 40), lambda k: (0, 0)),
        ],
        out_specs=pl.BlockSpec((B, 40), lambda k: (0, 0)),
        out_shape=jax.ShapeDtypeStruct((B, 40), jnp.float32),
        scratch_shapes=[pltpu.VMEM((B, 512), jnp.float32)],
    )(gf, wf1, s3, bf1f, wf2s, bf2f, wf3, bf3)


def kernel(points, w1, b1, g1, be1, w2, b2, g2, be2,
           wf1, bf1, g3, be3, wf2, bf2, g4, be4, wf3, bf3):
    s1 = g1 * jax.lax.rsqrt(1.0 + EPS)
    w1f = w1 * s1[None, :]
    b1f = (b1 * s1 + be1)[None, :]
    s2 = g2 * jax.lax.rsqrt(1.0 + EPS)
    w2f = w2 * s2[None, :]
    b2f = (b2 * s2 + be2)[None, :]
    s3 = (g3 * jax.lax.rsqrt(1.0 + EPS))[None, :]
    bf1f = (bf1 * s3[0] + be3)[None, :]
    s4 = g4 * jax.lax.rsqrt(1.0 + EPS)
    wf2s = wf2 * s4[None, :]
    bf2f = (bf2 * s4 + be4)[None, :]

    points2d = points.reshape(B * N, 3)
    feat = _point_mlp(points2d, w1f, b1f, w2f, b2f)

    normalized = (points + 1.0) / 2.0
    grid_idx = (normalized * (GRID - 1e-5)).astype(jnp.int32)
    grid_idx = jnp.clip(grid_idx, 0, GRID - 1)
    flat_idx = (grid_idx[..., 0] * GRID * GRID + grid_idx[..., 1] * GRID
                + grid_idx[..., 2])
    seg = (flat_idx + jnp.arange(B, dtype=jnp.int32)[:, None] * GV).reshape(-1)
    gf = jax.ops.segment_max(feat, seg, num_segments=B * GV)
    gf = jnp.where(jnp.isinf(gf), jnp.zeros_like(gf), gf)
    gf = gf.reshape(B, GV * F)

    return _head(gf, wf1, s3, bf1f, wf2s, bf2f, wf3, bf3[None, :])


# speedup vs baseline: 1.8440x; 1.8440x over previous
import functools

import jax
import jax.numpy as jnp
from jax import lax
from jax.experimental import pallas as pl
from jax.experimental.pallas import tpu as pltpu
from jax.experimental.pallas import tpu_sc as plsc

GRID = 8
GV = GRID ** 3
EPS = 1e-5
B = 32
N = 4096
F = 128


def _mlp_body(pts_ref, w1_ref, b1_ref, w2_ref, b2_ref, feat_ref):
    p = pts_ref[...]
    x = jnp.dot(p, w1_ref[...], preferred_element_type=jnp.float32)
    x = jnp.maximum(x + b1_ref[...], 0.0)
    y = jnp.dot(x, w2_ref[...], preferred_element_type=jnp.float32)
    feat_ref[...] = jnp.maximum(y + b2_ref[...], 0.0)


def _point_mlp(points2d, w1f, b1f, w2f, b2f):
    return pl.pallas_call(
        _mlp_body,
        grid=(B,),
        in_specs=[
            pl.BlockSpec((N, 3), lambda b: (b, 0)),
            pl.BlockSpec((3, 64), lambda b: (0, 0)),
            pl.BlockSpec((1, 64), lambda b: (0, 0)),
            pl.BlockSpec((64, 128), lambda b: (0, 0)),
            pl.BlockSpec((1, 128), lambda b: (0, 0)),
        ],
        out_specs=pl.BlockSpec((N, F), lambda b: (b, 0)),
        out_shape=jax.ShapeDtypeStruct((B * N, F), jnp.float32),
    )(points2d, w1f, b1f, w2f, b2f)


PCHUNK = 256
NCHUNK = N // PCHUNK
LANES = 16
FG = F // LANES


def _scatter_body(ptst_ref, feat_ref, gf_ref, pts_v, seg_v, gf_v, feat_v):
    wid = lax.axis_index("s") * 2 + lax.axis_index("c")

    pltpu.sync_copy(ptst_ref.at[wid], pts_v)

    cmax = jnp.full((LANES,), GRID - 1, jnp.int32)
    czero = jnp.zeros((LANES,), jnp.int32)
    scale = jnp.full((LANES,), jnp.float32(GRID - 1e-5), jnp.float32)

    def _seg_step(i, _):
        base = i * LANES
        def cell(comp):
            v = pts_v[pl.ds(comp * N + base, LANES)]
            t = ((v + 1.0) * 0.5) * scale
            return jnp.minimum(jnp.maximum(t.astype(jnp.int32), czero), cmax)
        c = (cell(0) * (GRID * GRID) + cell(1) * GRID + cell(2)) * F
        seg_v[pl.ds(base, LANES)] = c
        return _

    lax.fori_loop(0, N // LANES, _seg_step, None)

    zeros = jnp.zeros((LANES,), jnp.float32)

    def _init_step(i, _):
        for j in range(FG):
            gf_v[pl.ds(i * F + j * LANES, LANES)] = zeros
        return _

    lax.fori_loop(0, GV, _init_step, None)

    def _chunk_step(k, _):
        pltpu.sync_copy(
            feat_ref.at[pl.ds((wid * N + k * PCHUNK) * F, PCHUNK * F)],
            feat_v)

        def _group_step(g, _):
            segs = seg_v[pl.ds(k * PCHUNK + g * LANES, LANES)]
            for t in range(LANES):
                c = segs[t]
                pbase = (g * LANES + t) * F
                for j in range(FG):
                    f = feat_v[pl.ds(pbase + j * LANES, LANES)]
                    gcur = gf_v[pl.ds(c + j * LANES, LANES)]
                    gf_v[pl.ds(c + j * LANES, LANES)] = jnp.maximum(gcur, f)
            return _

        lax.fori_loop(0, PCHUNK // LANES, _group_step, None)
        return _

    lax.fori_loop(0, NCHUNK, _chunk_step, None)

    pltpu.sync_copy(gf_v, gf_ref.at[wid])


def _sc_grid_pool(ptst, feat):
    return pl.kernel(
        _scatter_body,
        out_type=jax.ShapeDtypeStruct((B, GV * F), jnp.float32),
        mesh=plsc.VectorSubcoreMesh(core_axis_name="c", subcore_axis_name="s"),
        scratch_types=[
            pltpu.VMEM((3 * N,), jnp.float32),
            pltpu.VMEM((N,), jnp.int32),
            pltpu.VMEM((GV * F,), jnp.float32),
            pltpu.VMEM((PCHUNK * F,), jnp.float32),
        ],
    )(ptst, feat)


KBLK = 4096
NKB = GV * F // KBLK


def _head_body(gf_ref, wf1_ref, s3_ref, bf1f_ref, wf2s_ref, bf2f_ref,
               wf3_ref, bf3_ref, out_ref, acc_ref):
    k = pl.program_id(0)

    @pl.when(k == 0)
    def _():
        acc_ref[...] = jnp.zeros_like(acc_ref)

    acc_ref[...] += jnp.dot(gf_ref[...], wf1_ref[...],
                            preferred_element_type=jnp.float32)

    @pl.when(k == NKB - 1)
    def _():
        h = jnp.maximum(acc_ref[...] * s3_ref[...] + bf1f_ref[...], 0.0)
        h2 = jnp.dot(h, wf2s_ref[...], preferred_element_type=jnp.float32)
        h2 = jnp.maximum(h2 + bf2f_ref[...], 0.0)
        out_ref[...] = jnp.dot(h2, wf3_ref[...],
                               preferred_element_type=jnp.float32) + bf3_ref[...]


def _head(gf, wf1, s3, bf1f, wf2s, bf2f, wf3, bf3):
    return pl.pallas_call(
        _head_body,
        grid=(NKB,),
        in_specs=[
            pl.BlockSpec((B, KBLK), lambda k: (0, k)),
            pl.BlockSpec((KBLK, 512), lambda k: (k, 0)),
            pl.BlockSpec((1, 512), lambda k: (0, 0)),
            pl.BlockSpec((1, 512), lambda k: (0, 0)),
            pl.BlockSpec((512, 256), lambda k: (0, 0)),
            pl.BlockSpec((1, 256), lambda k: (0, 0)),
            pl.BlockSpec((256, 40), lambda k: (0, 0)),
            pl.BlockSpec((1, 40), lambda k: (0, 0)),
        ],
        out_specs=pl.BlockSpec((B, 40), lambda k: (0, 0)),
        out_shape=jax.ShapeDtypeStruct((B, 40), jnp.float32),
        scratch_shapes=[pltpu.VMEM((B, 512), jnp.float32)],
    )(gf, wf1, s3, bf1f, wf2s, bf2f, wf3, bf3)


def kernel(points, w1, b1, g1, be1, w2, b2, g2, be2,
           wf1, bf1, g3, be3, wf2, bf2, g4, be4, wf3, bf3):
    s1 = g1 * jax.lax.rsqrt(1.0 + EPS)
    w1f = w1 * s1[None, :]
    b1f = (b1 * s1 + be1)[None, :]
    s2 = g2 * jax.lax.rsqrt(1.0 + EPS)
    w2f = w2 * s2[None, :]
    b2f = (b2 * s2 + be2)[None, :]
    s3 = (g3 * jax.lax.rsqrt(1.0 + EPS))[None, :]
    bf1f = (bf1 * s3[0] + be3)[None, :]
    s4 = g4 * jax.lax.rsqrt(1.0 + EPS)
    wf2s = wf2 * s4[None, :]
    bf2f = (bf2 * s4 + be4)[None, :]

    points2d = points.reshape(B * N, 3)
    feat = _point_mlp(points2d, w1f, b1f, w2f, b2f)

    ptst = points.transpose(0, 2, 1).reshape(B, 3 * N)
    gf = _sc_grid_pool(ptst, feat.reshape(-1))

    return _head(gf, wf1, s3, bf1f, wf2s, bf2f, wf3, bf3[None, :])
